# baseline (speedup 1.0000x reference)
import jax
import jax.numpy as jnp
from jax.experimental import pallas as pl

_NA = 3
_NC = 80
_NO = _NC + 5


def _transpose_body(in_ref, out_ref):
    out_ref[...] = jnp.transpose(in_ref[...], (0, 2, 1))


def kernel(p):
    bs, c, ny, nx = p.shape
    s = ny * nx
    x = p.reshape(bs * _NA, _NO, s)

    ts = 512
    out = pl.pallas_call(
        _transpose_body,
        grid=(bs * _NA, s // ts),
        in_specs=[pl.BlockSpec((1, _NO, ts), lambda i, j: (i, 0, j))],
        out_specs=pl.BlockSpec((1, ts, _NO), lambda i, j: (i, j, 0)),
        out_shape=jax.ShapeDtypeStruct((bs * _NA, s, _NO), jnp.float32),
    )(x)
    return out.reshape(bs, _NA, ny, nx, _NO)


# speedup vs baseline: 1.9088x; 1.9088x over previous
import jax
import jax.numpy as jnp
from jax.experimental import pallas as pl

_NA = 3
_NC = 80
_NO = _NC + 5


def _transpose_body(in_ref, out_ref):
    out_ref[...] = jnp.transpose(in_ref[...], (0, 2, 1))


def kernel(p):
    bs, c, ny, nx = p.shape
    s = ny * nx
    x = p.reshape(bs * _NA, _NO, s)

    nb = 4
    out = pl.pallas_call(
        _transpose_body,
        grid=(bs * _NA // nb,),
        in_specs=[pl.BlockSpec((nb, _NO, s), lambda i: (i, 0, 0))],
        out_specs=pl.BlockSpec((nb, s, _NO), lambda i: (i, 0, 0)),
        out_shape=jax.ShapeDtypeStruct((bs * _NA, s, _NO), jnp.float32),
    )(x)
    return out.reshape(bs, _NA, ny, nx, _NO)


# speedup vs baseline: 1.9243x; 1.0081x over previous
import jax
import jax.numpy as jnp
from jax.experimental import pallas as pl

_NA = 3
_NC = 80
_NO = _NC + 5


def _transpose_body(in_ref, out_ref):
    out_ref[...] = jnp.transpose(in_ref[...], (0, 2, 1))


def kernel(p):
    bs, c, ny, nx = p.shape
    s = ny * nx
    x = p.reshape(bs * _NA, _NO, s)

    nb = 8
    out = pl.pallas_call(
        _transpose_body,
        grid=(bs * _NA // nb,),
        in_specs=[pl.BlockSpec((nb, _NO, s), lambda i: (i, 0, 0))],
        out_specs=pl.BlockSpec((nb, s, _NO), lambda i: (i, 0, 0)),
        out_shape=jax.ShapeDtypeStruct((bs * _NA, s, _NO), jnp.float32),
    )(x)
    return out.reshape(bs, _NA, ny, nx, _NO)
